# Initial kernel scaffold; baseline (speedup 1.0000x reference)
import jax
import jax.numpy as jnp
from jax.experimental import pallas as pl


def kernel(x_intra, edge_index_intra, edge_attr_intra, batch_ei_intra, x_inter, edge_index_inter, edge_attr_inter, batch_ei_inter, gamma_i, beta_i, Wx_i, bx_i, We_i, asrc_i, adst_i, ae_i, bout_i, gamma_j, beta_j, Wx_j, bx_j, We_j, asrc_j, adst_j, ae_j, bout_j, Wg, bg, W1, W2):
    raise NotImplementedError("write your pallas kernel here")


# speedup vs baseline: 5.2373x; 5.2373x over previous
import functools

import jax
import jax.numpy as jnp
import numpy as np
from jax import lax
from jax.experimental import pallas as pl
from jax.experimental.pallas import tpu as pltpu
from jax.experimental.pallas import tpu_sc as plsc

_N = 10000
_E = 320000
_D = 128
_DE = 16
_H = 8
_DH = _D // _H
_K = int(np.ceil(0.8 * _E))

_NC = 2
_NS = 16
_NW = _NC * _NS


@functools.lru_cache(maxsize=None)
def _sc_gather_rows(V, Dw, B, dtype_name, W):
    dtype = jnp.dtype(dtype_name)
    b_per_w = B // _NW
    nwin = b_per_w // W
    assert b_per_w % W == 0 and W % 8 == 0 and W <= 128
    mesh = plsc.VectorSubcoreMesh(core_axis_name="c", subcore_axis_name="s")

    @functools.partial(
        pl.kernel,
        out_type=jax.ShapeDtypeStruct((B, Dw), dtype),
        mesh=mesh,
        compiler_params=pltpu.CompilerParams(use_tc_tiling_on_sc=(Dw % 128 == 0)),
        scratch_types=[
            pltpu.VMEM((W,), jnp.int32),
            pltpu.VMEM((W, Dw), dtype),
            pltpu.SemaphoreType.DMA,
        ],
    )
    def k(table_hbm, idx_hbm, out_hbm, idx_v, rows_v, sem):
        wid = lax.axis_index("s") * _NC + lax.axis_index("c")

        def body(w, carry):
            base = wid * b_per_w + w * W
            pltpu.sync_copy(idx_hbm.at[pl.ds(base, W)], idx_v)
            pltpu.async_copy(table_hbm.at[idx_v], rows_v, sem).wait()
            pltpu.sync_copy(rows_v, out_hbm.at[pl.ds(base, W)])
            return carry

        lax.fori_loop(0, nwin, body, 0)

    return k


@functools.lru_cache(maxsize=None)
def _sc_scatter_add_rows(V, Dw, B, W):
    b_per_w = B // _NW
    nwin = b_per_w // W
    assert b_per_w % W == 0 and W % 8 == 0 and W <= 128
    mesh = plsc.VectorSubcoreMesh(core_axis_name="c", subcore_axis_name="s")

    @functools.partial(
        pl.kernel,
        out_type=jax.ShapeDtypeStruct((_NC, V, Dw), jnp.float32),
        mesh=mesh,
        compiler_params=pltpu.CompilerParams(use_tc_tiling_on_sc=(Dw % 128 == 0)),
        scratch_types=[
            pltpu.VMEM((W,), jnp.int32),
            pltpu.VMEM((W, Dw), jnp.float32),
            pltpu.VMEM_SHARED((V, Dw), jnp.float32),
            pltpu.SemaphoreType.DMA,
        ],
    )
    def k(upd_hbm, idx_hbm, zero_hbm, out_hbm, idx_v, upd_v, acc_sh, sem):
        cid = lax.axis_index("c")
        sid = lax.axis_index("s")
        wid = sid * _NC + cid

        @pl.when(sid == 0)
        def _():
            pltpu.sync_copy(zero_hbm, acc_sh)

        plsc.subcore_barrier()

        def body(w, carry):
            base = wid * b_per_w + w * W
            pltpu.sync_copy(idx_hbm.at[pl.ds(base, W)], idx_v)
            pltpu.sync_copy(upd_hbm.at[pl.ds(base, W)], upd_v)
            pltpu.sync_copy(upd_v, acc_sh.at[idx_v], add=True)
            return carry

        lax.fori_loop(0, nwin, body, 0)
        plsc.subcore_barrier()

        @pl.when(sid == 0)
        def _():
            pltpu.sync_copy(acc_sh, out_hbm.at[cid])

    return k


_SR = 4096
_SC_ = 128
_S = _SR * _SC_
_NBITS = 19


def _sort_schedule():
    ds, sb = [], []
    for s in range(1, _NBITS + 1):
        d = 1 << (s - 1)
        while d >= 1:
            ds.append(d)
            sb.append(1 << s)
            d //= 2
    return np.array(ds, np.int32), np.array(sb, np.int32)


def _sort_body(score_ref, dsched_ref, out_ref, key_ref):
    rows = lax.broadcasted_iota(jnp.int32, (_SR, _SC_), 0)
    cols = lax.broadcasted_iota(jnp.int32, (_SR, _SC_), 1)
    ig = rows + _SR * cols
    b = pltpu.bitcast(score_ref[...], jnp.int32)
    key = jnp.where(b >= 0, jnp.int32(0x7FFFFFFF) - b, b) ^ jnp.int32(-2147483648)
    key_ref[...] = key
    out_ref[...] = ig

    nsteps = dsched_ref.shape[0] // 2

    def step(t, carry):
        d = dsched_ref[2 * t]
        sblk = dsched_ref[2 * t + 1]
        ai = key_ref[...]
        ix = out_ref[...]
        first = (ig & d) == 0
        asc = (ig & sblk) == 0
        keep_small = first == asc

        def row_case(ai, ix):
            return (
                pltpu.roll(ai, _SR - d, 0), pltpu.roll(ai, d, 0),
                pltpu.roll(ix, _SR - d, 0), pltpu.roll(ix, d, 0),
            )

        def col_case(ai, ix):
            m = d >> 12
            return (
                pltpu.roll(ai, _SC_ - m, 1), pltpu.roll(ai, m, 1),
                pltpu.roll(ix, _SC_ - m, 1), pltpu.roll(ix, m, 1),
            )

        fa, ba, fi, bi = lax.cond(d < _SR, row_case, col_case, ai, ix)
        pa = jnp.where(first, fa, ba)
        pi = jnp.where(first, fi, bi)
        mine_less = (ai < pa) | ((ai == pa) & (ix < pi))
        take = keep_small ^ mine_less
        key_ref[...] = jnp.where(take, pa, ai)
        out_ref[...] = jnp.where(take, pi, ix)
        return carry

    lax.fori_loop(0, nsteps, step, 0)


def _bitonic_argsort(score):
    pad = jnp.full((_S - _E,), -jnp.inf, jnp.float32)
    s2 = jnp.concatenate([score, pad]).reshape(_SC_, _SR).T
    ds, sb = _sort_schedule()
    sched = jnp.asarray(np.stack([ds, sb], 1).reshape(-1))
    idx2d, _ = pl.pallas_call(
        _sort_body,
        out_shape=(
            jax.ShapeDtypeStruct((_SR, _SC_), jnp.int32),
            jax.ShapeDtypeStruct((_SR, _SC_), jnp.int32),
        ),
        in_specs=[
            pl.BlockSpec(memory_space=pltpu.VMEM),
            pl.BlockSpec(memory_space=pltpu.SMEM),
        ],
        out_specs=(
            pl.BlockSpec(memory_space=pltpu.VMEM),
            pl.BlockSpec(memory_space=pltpu.VMEM),
        ),
    )(s2, sched)
    return idx2d.T.reshape(-1)


_BE2 = 8000


def _edge2_body(hsrc_ref, p_ref, g0_ref, g1_ref, rep_ref, out_ref):
    denom = g0_ref[...] + g1_ref[...] + jnp.float32(1e-16)
    alpha16 = p_ref[...] / denom
    afull = jnp.dot(alpha16, rep_ref[...], preferred_element_type=jnp.float32)
    out_ref[...] = hsrc_ref[...] * afull


def _edge2(hsrc, p16, gs0, gs1, rep):
    grid = _E // _BE2
    return pl.pallas_call(
        _edge2_body,
        grid=(grid,),
        in_specs=[
            pl.BlockSpec((_BE2, _D), lambda i: (i, 0)),
            pl.BlockSpec((_BE2, 16), lambda i: (i, 0)),
            pl.BlockSpec((_BE2, 16), lambda i: (i, 0)),
            pl.BlockSpec((_BE2, 16), lambda i: (i, 0)),
            pl.BlockSpec((16, _D), lambda i: (0, 0)),
        ],
        out_specs=pl.BlockSpec((_BE2, _D), lambda i: (i, 0)),
        out_shape=jax.ShapeDtypeStruct((_E, _D), jnp.float32),
    )(hsrc, p16, gs0, gs1, rep)


_BNF = 2000


def _final_body(ai_ref, bi_ref, aj_ref, bj_ref, wg_ref, bg_ref, w1_ref, w2_ref,
                out_ref):
    xi = ai_ref[0] + ai_ref[1] + bi_ref[...]
    xj = aj_ref[0] + aj_ref[1] + bj_ref[...]
    cat = jnp.concatenate([xi, xj], axis=1)
    g = jax.nn.sigmoid(
        jnp.dot(cat, wg_ref[...], preferred_element_type=jnp.float32)
        + bg_ref[...])
    fusion = (g * jnp.dot(xi, w1_ref[...], preferred_element_type=jnp.float32)
              + (1.0 - g) * jnp.dot(xj, w2_ref[...],
                                    preferred_element_type=jnp.float32))
    out_ref[0] = fusion + xi
    out_ref[1] = fusion + xj


def _final(acc_i, bout_i, acc_j, bout_j, Wg, bg, W1, W2):
    grid = _N // _BNF
    return pl.pallas_call(
        _final_body,
        grid=(grid,),
        in_specs=[
            pl.BlockSpec((2, _BNF, _D), lambda i: (0, i, 0)),
            pl.BlockSpec((1, _D), lambda i: (0, 0)),
            pl.BlockSpec((2, _BNF, _D), lambda i: (0, i, 0)),
            pl.BlockSpec((1, _D), lambda i: (0, 0)),
            pl.BlockSpec((2 * _D, _D), lambda i: (0, 0)),
            pl.BlockSpec((1, _D), lambda i: (0, 0)),
            pl.BlockSpec((_D, _D), lambda i: (0, 0)),
            pl.BlockSpec((_D, _D), lambda i: (0, 0)),
        ],
        out_specs=pl.BlockSpec((2, _BNF, _D), lambda i: (0, i, 0)),
        out_shape=jax.ShapeDtypeStruct((2, _N, _D), jnp.float32),
    )(acc_i, bout_i.reshape(1, _D), acc_j, bout_j.reshape(1, _D),
      Wg, bg.reshape(1, _D), W1, W2)


def _gat_branch(x, ei, ea, gamma, beta, Wx, bx, We, asrc, adst, ae, rep16,
                zeros16, zeros128):
    src = ei[0]
    dst = ei[1]

    mu = jnp.mean(x, axis=0)
    var = jnp.var(x, axis=0)
    xn = (x - mu) / jnp.sqrt(var + 1e-5) * gamma + beta
    h = (xn @ Wx + bx)
    h3 = h.reshape(_N, _H, _DH)
    he = (ea @ We).reshape(_E, _H, _DH)
    logits = jax.nn.leaky_relu(
        jnp.sum(h3[src] * asrc, -1) + jnp.sum(h3[dst] * adst, -1)
        + jnp.sum(he * ae, -1), 0.2)
    score = jnp.mean(logits, axis=-1)

    p8 = jnp.exp(logits)
    p16 = jnp.concatenate([p8, p8], axis=1)
    ssum = _sc_scatter_add_rows(_N, 16, _E, 80)(p16, dst, zeros16)
    gs0 = _sc_gather_rows(_N, 16, _E, "float32", 80)(ssum[0], dst)
    gs1 = _sc_gather_rows(_N, 16, _E, "float32", 80)(ssum[1], dst)

    hsrc = _sc_gather_rows(_N, _D, _E, "float32", 80)(h, src)
    upd = _edge2(hsrc, p16, gs0, gs1, rep16)
    acc = _sc_scatter_add_rows(_N, _D, _E, 80)(upd, dst, zeros128)

    return acc, score


def kernel(x_intra, edge_index_intra, edge_attr_intra, batch_ei_intra,
           x_inter, edge_index_inter, edge_attr_inter, batch_ei_inter,
           gamma_i, beta_i, Wx_i, bx_i, We_i, asrc_i, adst_i, ae_i, bout_i,
           gamma_j, beta_j, Wx_j, bx_j, We_j, asrc_j, adst_j, ae_j, bout_j,
           Wg, bg, W1, W2):
    rep16 = np.zeros((16, _D), np.float32)
    for hh in range(_H):
        rep16[hh, hh * _DH:(hh + 1) * _DH] = 1.0
    rep16 = jnp.asarray(rep16)
    zeros16 = jnp.zeros((_N, 16), jnp.float32)
    zeros128 = jnp.zeros((_N, _D), jnp.float32)

    acc_i, score_i = _gat_branch(
        x_intra, edge_index_intra, edge_attr_intra,
        gamma_i, beta_i, Wx_i, bx_i, We_i, asrc_i, adst_i, ae_i,
        rep16, zeros16, zeros128)
    acc_j, score_j = _gat_branch(
        x_inter, edge_index_inter, edge_attr_inter,
        gamma_j, beta_j, Wx_j, bx_j, We_j, asrc_j, adst_j, ae_j,
        rep16, zeros16, zeros128)

    out = _final(acc_i, bout_i, acc_j, bout_j, Wg, bg, W1, W2)

    idx_i = _bitonic_argsort(score_i)[:_K]
    idx_j = _bitonic_argsort(score_j)[:_K]
    pad14_i = jnp.concatenate(
        [edge_index_intra.T.astype(jnp.int32),
         jnp.zeros((_E, 14), jnp.int32)], axis=1)
    pad14_j = jnp.concatenate(
        [edge_index_inter.T.astype(jnp.int32),
         jnp.zeros((_E, 14), jnp.int32)], axis=1)
    ei_i = _sc_gather_rows(_E, 16, _K, "int32", 80)(pad14_i, idx_i)[:, :2].T
    ei_j = _sc_gather_rows(_E, 16, _K, "int32", 80)(pad14_j, idx_j)[:, :2].T

    return (out, ei_i, ei_j)


# speedup vs baseline: 6.0129x; 1.1481x over previous
import functools

import jax
import jax.numpy as jnp
import numpy as np
from jax import lax
from jax.experimental import pallas as pl
from jax.experimental.pallas import tpu as pltpu
from jax.experimental.pallas import tpu_sc as plsc

_N = 10000
_E = 320000
_D = 128
_DE = 16
_H = 8
_DH = _D // _H
_K = int(np.ceil(0.8 * _E))

_NC = 2
_NS = 16
_NW = _NC * _NS


@functools.lru_cache(maxsize=None)
def _sc_gather_rows(V, Dw, B, dtype_name, W):
    dtype = jnp.dtype(dtype_name)
    b_per_w = B // _NW
    nwin = b_per_w // W
    assert b_per_w % W == 0 and W % 8 == 0 and W <= 128
    mesh = plsc.VectorSubcoreMesh(core_axis_name="c", subcore_axis_name="s")

    @functools.partial(
        pl.kernel,
        out_type=jax.ShapeDtypeStruct((B, Dw), dtype),
        mesh=mesh,
        compiler_params=pltpu.CompilerParams(use_tc_tiling_on_sc=(Dw % 128 == 0)),
        scratch_types=[
            pltpu.VMEM((W,), jnp.int32),
            pltpu.VMEM((W, Dw), dtype),
            pltpu.SemaphoreType.DMA,
        ],
    )
    def k(table_hbm, idx_hbm, out_hbm, idx_v, rows_v, sem):
        wid = lax.axis_index("s") * _NC + lax.axis_index("c")

        def body(w, carry):
            base = wid * b_per_w + w * W
            pltpu.sync_copy(idx_hbm.at[pl.ds(base, W)], idx_v)
            pltpu.async_copy(table_hbm.at[idx_v], rows_v, sem).wait()
            pltpu.sync_copy(rows_v, out_hbm.at[pl.ds(base, W)])
            return carry

        lax.fori_loop(0, nwin, body, 0)

    return k


@functools.lru_cache(maxsize=None)
def _sc_scatter_add_rows(V, Dw, B, W):
    b_per_w = B // _NW
    nwin = b_per_w // W
    assert b_per_w % W == 0 and W % 8 == 0 and W <= 128
    mesh = plsc.VectorSubcoreMesh(core_axis_name="c", subcore_axis_name="s")

    @functools.partial(
        pl.kernel,
        out_type=jax.ShapeDtypeStruct((_NC, V, Dw), jnp.float32),
        mesh=mesh,
        compiler_params=pltpu.CompilerParams(use_tc_tiling_on_sc=(Dw % 128 == 0)),
        scratch_types=[
            pltpu.VMEM((W,), jnp.int32),
            pltpu.VMEM((W, Dw), jnp.float32),
            pltpu.VMEM_SHARED((V, Dw), jnp.float32),
            pltpu.SemaphoreType.DMA,
        ],
    )
    def k(upd_hbm, idx_hbm, zero_hbm, out_hbm, idx_v, upd_v, acc_sh, sem):
        cid = lax.axis_index("c")
        sid = lax.axis_index("s")
        wid = sid * _NC + cid

        @pl.when(sid == 0)
        def _():
            pltpu.sync_copy(zero_hbm, acc_sh)

        plsc.subcore_barrier()

        def body(w, carry):
            base = wid * b_per_w + w * W
            pltpu.sync_copy(idx_hbm.at[pl.ds(base, W)], idx_v)
            pltpu.sync_copy(upd_hbm.at[pl.ds(base, W)], upd_v)
            pltpu.sync_copy(upd_v, acc_sh.at[idx_v], add=True)
            return carry

        lax.fori_loop(0, nwin, body, 0)
        plsc.subcore_barrier()

        @pl.when(sid == 0)
        def _():
            pltpu.sync_copy(acc_sh, out_hbm.at[cid])

    return k


_SR = 4096
_SC_ = 128
_S = _SR * _SC_
_NBITS = 19


def _sort_schedule():
    ds, sb = [], []
    for s in range(1, _NBITS + 1):
        d = 1 << (s - 1)
        while d >= 1:
            ds.append(d)
            sb.append(1 << s)
            d //= 2
    return np.array(ds, np.int32), np.array(sb, np.int32)


def _sort_body(score_ref, dsched_ref, out_ref, key_ref):
    rows = lax.broadcasted_iota(jnp.int32, (_SR, _SC_), 0)
    cols = lax.broadcasted_iota(jnp.int32, (_SR, _SC_), 1)
    ig = rows + _SR * cols
    b = pltpu.bitcast(score_ref[...], jnp.int32)
    key = jnp.where(b >= 0, jnp.int32(0x7FFFFFFF) - b, b) ^ jnp.int32(-2147483648)
    key_ref[...] = key
    out_ref[...] = ig

    nsteps = dsched_ref.shape[0] // 2

    def step(t, carry):
        d = dsched_ref[2 * t]
        sblk = dsched_ref[2 * t + 1]
        ai = key_ref[...]
        ix = out_ref[...]
        first = (ig & d) == 0
        asc = (ig & sblk) == 0
        keep_small = first == asc

        def row_case(ai, ix):
            return (
                pltpu.roll(ai, _SR - d, 0), pltpu.roll(ai, d, 0),
                pltpu.roll(ix, _SR - d, 0), pltpu.roll(ix, d, 0),
            )

        def col_case(ai, ix):
            m = d >> 12
            return (
                pltpu.roll(ai, _SC_ - m, 1), pltpu.roll(ai, m, 1),
                pltpu.roll(ix, _SC_ - m, 1), pltpu.roll(ix, m, 1),
            )

        fa, ba, fi, bi = lax.cond(d < _SR, row_case, col_case, ai, ix)
        pa = jnp.where(first, fa, ba)
        pi = jnp.where(first, fi, bi)
        mine_less = (ai < pa) | ((ai == pa) & (ix < pi))
        take = keep_small ^ mine_less
        key_ref[...] = jnp.where(take, pa, ai)
        out_ref[...] = jnp.where(take, pi, ix)
        return carry

    lax.fori_loop(0, nsteps, step, 0)


def _bitonic_argsort(score):
    pad = jnp.full((_S - _E,), -jnp.inf, jnp.float32)
    s2 = jnp.concatenate([score, pad]).reshape(_SC_, _SR).T
    ds, sb = _sort_schedule()
    sched = jnp.asarray(np.stack([ds, sb], 1).reshape(-1))
    idx2d, _ = pl.pallas_call(
        _sort_body,
        out_shape=(
            jax.ShapeDtypeStruct((_SR, _SC_), jnp.int32),
            jax.ShapeDtypeStruct((_SR, _SC_), jnp.int32),
        ),
        in_specs=[
            pl.BlockSpec(memory_space=pltpu.VMEM),
            pl.BlockSpec(memory_space=pltpu.SMEM),
        ],
        out_specs=(
            pl.BlockSpec(memory_space=pltpu.VMEM),
            pl.BlockSpec(memory_space=pltpu.VMEM),
        ),
    )(s2, sched)
    return idx2d.T.reshape(-1)


_BE2 = 8000


def _edge2_body(hsrc_ref, p_ref, g0_ref, g1_ref, rep_ref, out_ref):
    denom = g0_ref[...] + g1_ref[...] + jnp.float32(1e-16)
    alpha16 = p_ref[...] / denom
    afull = jnp.dot(alpha16, rep_ref[...], preferred_element_type=jnp.float32)
    out_ref[...] = hsrc_ref[...] * afull


def _edge2(hsrc, p16, gs0, gs1, rep):
    grid = _E // _BE2
    return pl.pallas_call(
        _edge2_body,
        grid=(grid,),
        in_specs=[
            pl.BlockSpec((_BE2, _D), lambda i: (i, 0)),
            pl.BlockSpec((_BE2, 16), lambda i: (i, 0)),
            pl.BlockSpec((_BE2, 16), lambda i: (i, 0)),
            pl.BlockSpec((_BE2, 16), lambda i: (i, 0)),
            pl.BlockSpec((16, _D), lambda i: (0, 0)),
        ],
        out_specs=pl.BlockSpec((_BE2, _D), lambda i: (i, 0)),
        out_shape=jax.ShapeDtypeStruct((_E, _D), jnp.float32),
    )(hsrc, p16, gs0, gs1, rep)


_BNF = 2000


def _final_body(ai_ref, bi_ref, aj_ref, bj_ref, wg_ref, bg_ref, w1_ref, w2_ref,
                out_ref):
    xi = ai_ref[0] + ai_ref[1] + bi_ref[...]
    xj = aj_ref[0] + aj_ref[1] + bj_ref[...]
    cat = jnp.concatenate([xi, xj], axis=1)
    g = jax.nn.sigmoid(
        jnp.dot(cat, wg_ref[...], preferred_element_type=jnp.float32)
        + bg_ref[...])
    fusion = (g * jnp.dot(xi, w1_ref[...], preferred_element_type=jnp.float32)
              + (1.0 - g) * jnp.dot(xj, w2_ref[...],
                                    preferred_element_type=jnp.float32))
    out_ref[0] = fusion + xi
    out_ref[1] = fusion + xj


def _final(acc_i, bout_i, acc_j, bout_j, Wg, bg, W1, W2):
    grid = _N // _BNF
    return pl.pallas_call(
        _final_body,
        grid=(grid,),
        in_specs=[
            pl.BlockSpec((2, _BNF, _D), lambda i: (0, i, 0)),
            pl.BlockSpec((1, _D), lambda i: (0, 0)),
            pl.BlockSpec((2, _BNF, _D), lambda i: (0, i, 0)),
            pl.BlockSpec((1, _D), lambda i: (0, 0)),
            pl.BlockSpec((2 * _D, _D), lambda i: (0, 0)),
            pl.BlockSpec((1, _D), lambda i: (0, 0)),
            pl.BlockSpec((_D, _D), lambda i: (0, 0)),
            pl.BlockSpec((_D, _D), lambda i: (0, 0)),
        ],
        out_specs=pl.BlockSpec((2, _BNF, _D), lambda i: (0, i, 0)),
        out_shape=jax.ShapeDtypeStruct((2, _N, _D), jnp.float32),
    )(acc_i, bout_i.reshape(1, _D), acc_j, bout_j.reshape(1, _D),
      Wg, bg.reshape(1, _D), W1, W2)


def _tree_sum(t):
    while t.shape[-1] > 1:
        t = t[..., 0::2] + t[..., 1::2]
    return t[..., 0]


def _fold_sum(t):
    while t.shape[-1] > 1:
        m = t.shape[-1] // 2
        t = t[..., :m] + t[..., m:]
    return t[..., 0]


def _gat_branch(x, ei, ea, gamma, beta, Wx, bx, We, asrc, adst, ae, rep16,
                zeros16, zeros128):
    src = ei[0]
    dst = ei[1]

    mu = jnp.mean(x, axis=0)
    var = jnp.var(x, axis=0)
    xn = (x - mu) / jnp.sqrt(var + 1e-5) * gamma + beta
    h = (xn @ Wx + bx)
    hsrc = _sc_gather_rows(_N, _D, _E, "float32", 80)(h, src)
    hdst = _sc_gather_rows(_N, _D, _E, "float32", 80)(h, dst)
    t1 = _tree_sum(hsrc.reshape(_E, _H, _DH) * asrc)
    t2 = _tree_sum(hdst.reshape(_E, _H, _DH) * adst)
    he = (ea @ We).reshape(_E, _H, _DH)
    t3 = _fold_sum(he * ae)
    logits = jax.nn.leaky_relu((t1 + t2) + t3, 0.2)
    score = _fold_sum(logits) / jnp.float32(8.0)

    p8 = jnp.exp(logits)
    p16 = jnp.concatenate([p8, p8], axis=1)
    ssum = _sc_scatter_add_rows(_N, 16, _E, 80)(p16, dst, zeros16)
    gs0 = _sc_gather_rows(_N, 16, _E, "float32", 80)(ssum[0], dst)
    gs1 = _sc_gather_rows(_N, 16, _E, "float32", 80)(ssum[1], dst)

    upd = _edge2(hsrc, p16, gs0, gs1, rep16)
    acc = _sc_scatter_add_rows(_N, _D, _E, 80)(upd, dst, zeros128)

    return acc, score


def kernel(x_intra, edge_index_intra, edge_attr_intra, batch_ei_intra,
           x_inter, edge_index_inter, edge_attr_inter, batch_ei_inter,
           gamma_i, beta_i, Wx_i, bx_i, We_i, asrc_i, adst_i, ae_i, bout_i,
           gamma_j, beta_j, Wx_j, bx_j, We_j, asrc_j, adst_j, ae_j, bout_j,
           Wg, bg, W1, W2):
    rep16 = np.zeros((16, _D), np.float32)
    for hh in range(_H):
        rep16[hh, hh * _DH:(hh + 1) * _DH] = 1.0
    rep16 = jnp.asarray(rep16)
    zeros16 = jnp.zeros((_N, 16), jnp.float32)
    zeros128 = jnp.zeros((_N, _D), jnp.float32)

    acc_i, score_i = _gat_branch(
        x_intra, edge_index_intra, edge_attr_intra,
        gamma_i, beta_i, Wx_i, bx_i, We_i, asrc_i, adst_i, ae_i,
        rep16, zeros16, zeros128)
    acc_j, score_j = _gat_branch(
        x_inter, edge_index_inter, edge_attr_inter,
        gamma_j, beta_j, Wx_j, bx_j, We_j, asrc_j, adst_j, ae_j,
        rep16, zeros16, zeros128)

    out = _final(acc_i, bout_i, acc_j, bout_j, Wg, bg, W1, W2)

    idx_i = _bitonic_argsort(score_i)[:_K]
    idx_j = _bitonic_argsort(score_j)[:_K]
    pad14_i = jnp.concatenate(
        [edge_index_intra.T.astype(jnp.int32),
         jnp.zeros((_E, 14), jnp.int32)], axis=1)
    pad14_j = jnp.concatenate(
        [edge_index_inter.T.astype(jnp.int32),
         jnp.zeros((_E, 14), jnp.int32)], axis=1)
    ei_i = _sc_gather_rows(_E, 16, _K, "int32", 80)(pad14_i, idx_i)[:, :2].T
    ei_j = _sc_gather_rows(_E, 16, _K, "int32", 80)(pad14_j, idx_j)[:, :2].T

    return (out, ei_i, ei_j)
